# Initial kernel scaffold; baseline (speedup 1.0000x reference)
import jax
import jax.numpy as jnp
from jax.experimental import pallas as pl


def kernel(x, edge_index, batch, W1l, b1, W1r, W2l, b2, W2r, Wlin, blin):
    raise NotImplementedError("write your pallas kernel here")


# speedup vs baseline: 7.0257x; 7.0257x over previous
import jax
import jax.numpy as jnp
from jax import lax
from jax.experimental import pallas as pl
from jax.experimental.pallas import tpu as pltpu
from jax.experimental.pallas import tpu_sc as plsc

NC = 2
NS = 16
NW = NC * NS
EB = 128
BN = 2048


def _pre_body(x_ref, w_ref, y_ref, xr_ref):
    h = y_ref.shape[1]
    p = jnp.dot(x_ref[...], w_ref[...], preferred_element_type=jnp.float32)
    y_ref[...] = p[:, :h]
    xr_ref[...] = p[:, h:]


def _mid_body(p_ref, dT_ref, xr_ref, b1_ref, w2_ref, y2_ref, hr2_ref):
    h = y2_ref.shape[1]
    agg = p_ref[0] + p_ref[1]
    deg = dT_ref[:, 0:1] + dT_ref[:, 1:2]
    inv = 1.0 / jnp.maximum(deg, 1.0)
    h1 = jnp.maximum(agg * inv + b1_ref[0] + xr_ref[...], 0.0)
    q = jnp.dot(h1, w2_ref[...], preferred_element_type=jnp.float32)
    y2_ref[...] = q[:, :h]
    hr2_ref[...] = q[:, h:]


def _post_body(p_ref, dT_ref, hr2_ref, b2_ref, wl_ref, bl_ref, out_ref):
    agg = p_ref[0] + p_ref[1]
    deg = dT_ref[:, 0:1] + dT_ref[:, 1:2]
    inv = 1.0 / jnp.maximum(deg, 1.0)
    h2 = jnp.maximum(agg * inv + b2_ref[0] + hr2_ref[...], 0.0)
    out_ref[...] = (
        jnp.dot(h2, wl_ref[...], preferred_element_type=jnp.float32) + bl_ref[0]
    )


def _make_agg(np_, h, e, with_deg):
    ngroups = e // EB
    nr = np_ // NS
    mesh = plsc.VectorSubcoreMesh(core_axis_name="c", subcore_axis_name="s")
    out_type = [jax.ShapeDtypeStruct((NC, np_, h), jnp.float32)]
    scratch = [
        pltpu.VMEM((EB,), jnp.int32),
        pltpu.VMEM((EB,), jnp.int32),
        pltpu.VMEM((EB, h), jnp.float32),
        pltpu.VMEM_SHARED((np_, h), jnp.float32),
        pltpu.SemaphoreType.DMA,
    ]
    if with_deg:
        out_type.append(jax.ShapeDtypeStruct((NC, np_), jnp.float32))
        scratch += [
            pltpu.VMEM((EB,), jnp.float32),
            pltpu.VMEM_SHARED((np_,), jnp.float32),
        ]

    def body(y_hbm, src_hbm, dst_hbm, zacc_hbm, *rest):
        if with_deg:
            (zdeg_hbm, acc_out, deg_out,
             src_v, dst_v, rows_v, acc_sh, sem, ones_v, deg_sh) = rest
        else:
            (acc_out, src_v, dst_v, rows_v, acc_sh, sem) = rest
        c = lax.axis_index("c")
        s = lax.axis_index("s")
        w = s * NC + c

        pltpu.sync_copy(zacc_hbm.at[pl.ds(s * nr, nr)],
                        acc_sh.at[pl.ds(s * nr, nr)])
        if with_deg:
            @pl.when(s == 0)
            def _():
                pltpu.sync_copy(zdeg_hbm, deg_sh)
            for j in range(EB // 16):
                ones_v[pl.ds(j * 16, 16)] = jnp.full((16,), 1.0, jnp.float32)
        plsc.subcore_barrier()

        ngw = ngroups // NW + jnp.where(w < ngroups % NW, 1, 0).astype(jnp.int32)

        def grp(i, carry):
            base = (w + i * NW) * EB
            pltpu.sync_copy(src_hbm.at[pl.ds(base, EB)], src_v)
            pltpu.sync_copy(dst_hbm.at[pl.ds(base, EB)], dst_v)
            pltpu.async_copy(y_hbm.at[src_v], rows_v, sem).wait()
            pltpu.sync_copy(rows_v, acc_sh.at[dst_v], add=True)
            if with_deg:
                pltpu.sync_copy(ones_v, deg_sh.at[dst_v], add=True)
            return carry

        lax.fori_loop(0, ngw, grp, 0)
        plsc.subcore_barrier()

        pltpu.sync_copy(acc_sh.at[pl.ds(s * nr, nr)],
                        acc_out.at[c, pl.ds(s * nr, nr)])
        if with_deg:
            @pl.when(s == 0)
            def _():
                pltpu.sync_copy(deg_sh, deg_out.at[c])

    return pl.kernel(
        body, out_type=out_type, mesh=mesh, scratch_types=scratch,
        compiler_params=pltpu.CompilerParams(use_tc_tiling_on_sc=False),
    )


def kernel(x, edge_index, batch, W1l, b1, W1r, W2l, b2, W2r, Wlin, blin):
    n, f_in = x.shape
    e = edge_index.shape[1]
    h = W1l.shape[1]
    c_out = Wlin.shape[1]
    np_ = ((n + BN - 1) // BN) * BN
    nblk = np_ // BN

    xp = jnp.pad(x, ((0, np_ - n), (0, 0)))
    src = edge_index[0]
    dst = edge_index[1]
    zacc = jnp.zeros((np_, h), jnp.float32)
    zdeg = jnp.zeros((np_,), jnp.float32)

    wcat1 = jnp.concatenate([W1l, W1r], axis=1)
    y1, xr1 = pl.pallas_call(
        _pre_body,
        grid=(nblk,),
        in_specs=[
            pl.BlockSpec((BN, f_in), lambda i: (i, 0)),
            pl.BlockSpec((f_in, 2 * h), lambda i: (0, 0)),
        ],
        out_specs=[
            pl.BlockSpec((BN, h), lambda i: (i, 0)),
            pl.BlockSpec((BN, h), lambda i: (i, 0)),
        ],
        out_shape=[
            jax.ShapeDtypeStruct((np_, h), jnp.float32),
            jax.ShapeDtypeStruct((np_, h), jnp.float32),
        ],
    )(xp, wcat1)

    acc1, deg = _make_agg(np_, h, e, True)(y1, src, dst, zacc, zdeg)
    degT = deg.T

    wcat2 = jnp.concatenate([W2l, W2r], axis=1)
    y2, hr2 = pl.pallas_call(
        _mid_body,
        grid=(nblk,),
        in_specs=[
            pl.BlockSpec((NC, BN, h), lambda i: (0, i, 0)),
            pl.BlockSpec((BN, NC), lambda i: (i, 0)),
            pl.BlockSpec((BN, h), lambda i: (i, 0)),
            pl.BlockSpec((1, h), lambda i: (0, 0)),
            pl.BlockSpec((h, 2 * h), lambda i: (0, 0)),
        ],
        out_specs=[
            pl.BlockSpec((BN, h), lambda i: (i, 0)),
            pl.BlockSpec((BN, h), lambda i: (i, 0)),
        ],
        out_shape=[
            jax.ShapeDtypeStruct((np_, h), jnp.float32),
            jax.ShapeDtypeStruct((np_, h), jnp.float32),
        ],
    )(acc1, degT, xr1, b1.reshape(1, h), wcat2)

    (acc2,) = _make_agg(np_, h, e, False)(y2, src, dst, zacc)

    out = pl.pallas_call(
        _post_body,
        grid=(nblk,),
        in_specs=[
            pl.BlockSpec((NC, BN, h), lambda i: (0, i, 0)),
            pl.BlockSpec((BN, NC), lambda i: (i, 0)),
            pl.BlockSpec((BN, h), lambda i: (i, 0)),
            pl.BlockSpec((1, h), lambda i: (0, 0)),
            pl.BlockSpec((h, c_out), lambda i: (0, 0)),
            pl.BlockSpec((1, c_out), lambda i: (0, 0)),
        ],
        out_specs=pl.BlockSpec((BN, c_out), lambda i: (i, 0)),
        out_shape=jax.ShapeDtypeStruct((np_, c_out), jnp.float32),
    )(acc2, degT, hr2, b2.reshape(1, h), Wlin, blin.reshape(1, c_out))

    return out[:n]
